# baseline (speedup 1.0000x reference)
import functools

import jax
import jax.numpy as jnp
from jax import lax
from jax.experimental import pallas as pl
from jax.experimental.pallas import tpu as pltpu
from jax.experimental.pallas import tpu_sc as plsc

ROWS = 16384
COLS = 20
NC = 2
NS = 16
NW = NC * NS
RPW = ROWS // NW

_mesh = plsc.VectorSubcoreMesh(core_axis_name="c", subcore_axis_name="s")


@functools.partial(
    pl.kernel,
    mesh=_mesh,
    out_type=jax.ShapeDtypeStruct((ROWS, COLS), jnp.float32),
    scratch_types=[
        pltpu.VMEM((RPW, COLS), jnp.float32),
    ],
)
def _d4(logits_hbm, state_hbm, out_hbm, buf):
    wid = lax.axis_index("s") * NC + lax.axis_index("c")
    base = wid * RPW
    pltpu.sync_copy(logits_hbm.at[pl.ds(base, RPW), :], buf)
    pltpu.sync_copy(buf, out_hbm.at[pl.ds(base, RPW), :])


def kernel(logits, latest_state):
    return _d4(logits, latest_state)


# speedup vs baseline: 1.3153x; 1.3153x over previous
import functools

import jax
import jax.numpy as jnp
from jax import lax
from jax.experimental import pallas as pl
from jax.experimental.pallas import tpu as pltpu
from jax.experimental.pallas import tpu_sc as plsc

ROWS = 16384
COLS = 20
NC = 2
NS = 16
NW = NC * NS
RPW = ROWS // NW
CHUNK = RPW * COLS
SCHUNK = RPW * 3
GROUPS = RPW // 16

COMBOS = ((1, 2), (3, 11), (17, 18))

_mesh = plsc.VectorSubcoreMesh(core_axis_name="c", subcore_axis_name="s")


@functools.partial(
    pl.kernel,
    mesh=_mesh,
    out_type=jax.ShapeDtypeStruct((ROWS * COLS,), jnp.float32),
    scratch_types=[
        pltpu.VMEM((CHUNK,), jnp.float32),
        pltpu.VMEM((SCHUNK,), jnp.float32),
    ],
)
def _safety_sc(logits_hbm, state_hbm, out_hbm, buf, sbuf):
    wid = lax.axis_index("s") * NC + lax.axis_index("c")
    pltpu.sync_copy(logits_hbm.at[pl.ds(wid * CHUNK, CHUNK)], buf)
    pltpu.sync_copy(state_hbm.at[pl.ds(wid * SCHUNK, SCHUNK)], sbuf)

    def body(g, carry):
        off = g * 16

        v = [buf[pl.ds(c * RPW + off, 16)] for c in range(COLS)]
        m = v[0]
        for c in range(1, COLS):
            m = jnp.maximum(m, v[c])
        e = [jnp.exp(v[c] - m) for c in range(COLS)]
        s = e[0]
        for c in range(1, COLS):
            s = s + e[c]
        p = {c: e[c] / s for combo in COMBOS for c in combo}

        def w(mask, val):
            return jnp.where(mask, jnp.float32(val), jnp.float32(0.0))

        adj = {}
        for a, b in COMBOS:
            risk = w(p[a] * p[b] > 0.05, -1.5)
            adj[a] = risk
            adj[b] = risk

        hr = sbuf[pl.ds(off, 16)]
        bp = sbuf[pl.ds(RPW + off, 16)]
        o2 = sbuf[pl.ds(2 * RPW + off, 16)]
        adj[2] = adj[2] + w(bp < 85.0, -5.0)
        adj[1] = adj[1] + w(bp < 85.0, 0.5) + w(bp > 160.0, -3.0)
        adj[4] = w(hr > 130.0, 0.3)
        adj[5] = w(o2 < 90.0, 0.5)

        for c, a in adj.items():
            buf[pl.ds(c * RPW + off, 16)] = v[c] + a
        return carry

    lax.fori_loop(0, GROUPS, body, jnp.int32(0))
    pltpu.sync_copy(buf, out_hbm.at[pl.ds(wid * CHUNK, CHUNK)])


def kernel(logits, latest_state):
    lt = logits.reshape(NW, RPW, COLS).transpose(0, 2, 1).reshape(-1)
    sv = (
        latest_state[:, (1, 3, 5)]
        .reshape(NW, RPW, 3)
        .transpose(0, 2, 1)
        .reshape(-1)
    )
    out = _safety_sc(lt, sv)
    return out.reshape(NW, COLS, RPW).transpose(0, 2, 1).reshape(ROWS, COLS)


# speedup vs baseline: 1.4326x; 1.0891x over previous
import functools

import jax
import jax.numpy as jnp
from jax import lax
from jax.experimental import pallas as pl
from jax.experimental.pallas import tpu as pltpu
from jax.experimental.pallas import tpu_sc as plsc

ROWS = 16384
COLS = 20
NC = 2
NS = 16
NW = NC * NS
RPW = ROWS // NW
CHUNK = RPW * COLS
GROUPS = RPW // 16

COMBOS = ((1, 2), (3, 11), (17, 18))
MUTABLE = (1, 2, 3, 4, 5, 11, 17, 18)

_mesh = plsc.VectorSubcoreMesh(core_axis_name="c", subcore_axis_name="s")


@functools.partial(
    pl.kernel,
    mesh=_mesh,
    out_type=jax.ShapeDtypeStruct((COLS * ROWS,), jnp.float32),
    scratch_types=[
        pltpu.VMEM((CHUNK,), jnp.float32),
        pltpu.VMEM((3 * RPW,), jnp.float32),
        pltpu.SemaphoreType.DMA,
    ],
)
def _safety_sc(logits_hbm, state_hbm, out_hbm, buf, sbuf, sem):
    wid = lax.axis_index("s") * NC + lax.axis_index("c")
    base = wid * RPW

    cps = [
        pltpu.async_copy(
            logits_hbm.at[pl.ds(c * ROWS + base, RPW)],
            buf.at[pl.ds(c * RPW, RPW)],
            sem,
        )
        for c in range(COLS)
    ] + [
        pltpu.async_copy(
            state_hbm.at[pl.ds(r * ROWS + base, RPW)],
            sbuf.at[pl.ds(i * RPW, RPW)],
            sem,
        )
        for i, r in enumerate((1, 3, 5))
    ]
    for cp in cps:
        cp.wait()

    def group(off):
        v = [buf[pl.ds(c * RPW + off, 16)] for c in range(COLS)]
        m = v[0]
        for c in range(1, COLS):
            m = jnp.maximum(m, v[c])
        e = [jnp.exp(v[c] - m) for c in range(COLS)]
        s = e[0]
        for c in range(1, COLS):
            s = s + e[c]
        thr = (0.05 * s) * s

        def w(mask, val):
            return jnp.where(mask, jnp.float32(val), jnp.float32(0.0))

        adj = {}
        for a, b in COMBOS:
            risk = w(e[a] * e[b] > thr, -1.5)
            adj[a] = risk
            adj[b] = risk

        hr = sbuf[pl.ds(off, 16)]
        bp = sbuf[pl.ds(RPW + off, 16)]
        o2 = sbuf[pl.ds(2 * RPW + off, 16)]
        adj[2] = adj[2] + w(bp < 85.0, -5.0)
        adj[1] = adj[1] + w(bp < 85.0, 0.5) + w(bp > 160.0, -3.0)
        adj[4] = w(hr > 130.0, 0.3)
        adj[5] = w(o2 < 90.0, 0.5)

        for c, a in adj.items():
            buf[pl.ds(c * RPW + off, 16)] = v[c] + a

    def body(g, carry):
        group(g * 32)
        group(g * 32 + 16)
        return carry

    lax.fori_loop(0, GROUPS // 2, body, jnp.int32(0))

    ops = [
        pltpu.async_copy(
            buf.at[pl.ds(c * RPW, RPW)],
            out_hbm.at[pl.ds(c * ROWS + base, RPW)],
            sem,
        )
        for c in range(COLS)
    ]
    for cp in ops:
        cp.wait()


def kernel(logits, latest_state):
    lt = logits.T.reshape(-1)
    sv = latest_state.T.reshape(-1)
    out = _safety_sc(lt, sv)
    return out.reshape(COLS, ROWS).T


# speedup vs baseline: 1.7118x; 1.1949x over previous
import functools

import jax
import jax.numpy as jnp
from jax import lax
from jax.experimental import pallas as pl
from jax.experimental.pallas import tpu as pltpu
from jax.experimental.pallas import tpu_sc as plsc

ROWS = 16384
COLS = 20
NC = 2
NS = 16
NW = NC * NS
RPW = ROWS // NW
CHUNK = RPW * COLS
GROUPS = RPW // 16

COMBOS = ((1, 2), (3, 11), (17, 18))
MUTABLE = (1, 2, 3, 4, 5, 11, 17, 18)

_mesh = plsc.VectorSubcoreMesh(core_axis_name="c", subcore_axis_name="s")


@functools.partial(
    pl.kernel,
    mesh=_mesh,
    out_type=jax.ShapeDtypeStruct((COLS, ROWS), jnp.float32),
    scratch_types=[
        pltpu.VMEM((CHUNK,), jnp.float32),
        pltpu.VMEM((3 * RPW,), jnp.float32),
        pltpu.SemaphoreType.DMA,
    ],
)
def _safety_sc(logits_hbm, state_hbm, out_hbm, buf, sbuf, sem):
    wid = lax.axis_index("s") * NC + lax.axis_index("c")
    base = wid * RPW

    cps = [
        pltpu.async_copy(
            logits_hbm.at[c, pl.ds(base, RPW)],
            buf.at[pl.ds(c * RPW, RPW)],
            sem,
        )
        for c in range(COLS)
    ] + [
        pltpu.async_copy(
            state_hbm.at[pl.ds(r * ROWS + base, RPW)],
            sbuf.at[pl.ds(i * RPW, RPW)],
            sem,
        )
        for i, r in enumerate((1, 3, 5))
    ]
    for cp in cps:
        cp.wait()

    def group(off):
        v = [buf[pl.ds(c * RPW + off, 16)] for c in range(COLS)]
        m = v[0]
        for c in range(1, COLS):
            m = jnp.maximum(m, v[c])
        e = [jnp.exp(v[c] - m) for c in range(COLS)]
        s = e[0]
        for c in range(1, COLS):
            s = s + e[c]
        thr = (0.05 * s) * s

        def w(mask, val):
            return jnp.where(mask, jnp.float32(val), jnp.float32(0.0))

        adj = {}
        for a, b in COMBOS:
            risk = w(e[a] * e[b] > thr, -1.5)
            adj[a] = risk
            adj[b] = risk

        hr = sbuf[pl.ds(off, 16)]
        bp = sbuf[pl.ds(RPW + off, 16)]
        o2 = sbuf[pl.ds(2 * RPW + off, 16)]
        adj[2] = adj[2] + w(bp < 85.0, -5.0)
        adj[1] = adj[1] + w(bp < 85.0, 0.5) + w(bp > 160.0, -3.0)
        adj[4] = w(hr > 130.0, 0.3)
        adj[5] = w(o2 < 90.0, 0.5)

        for c, a in adj.items():
            buf[pl.ds(c * RPW + off, 16)] = v[c] + a

    def body(g, carry):
        group(g * 16)
        return carry

    lax.fori_loop(0, GROUPS, body, jnp.int32(0))

    ops = [
        pltpu.async_copy(
            buf.at[pl.ds(c * RPW, RPW)],
            out_hbm.at[c, pl.ds(base, RPW)],
            sem,
        )
        for c in range(COLS)
    ]
    for cp in ops:
        cp.wait()


def kernel(logits, latest_state):
    lt = logits.T
    sv = latest_state.T.reshape(-1)
    out = _safety_sc(lt, sv)
    return out.T
